# baseline (speedup 1.0000x reference)
import functools

import jax
import jax.numpy as jnp
from jax import lax
from jax.experimental import pallas as pl
from jax.experimental.pallas import tpu as pltpu
from jax.experimental.pallas import tpu_sc as plsc

B, T, D = 64, 2048, 64
VP = 500000
NC, NS = 2, 16
TS = 128
TP = TS // 2
BH = B // NC
BC = 2
NCH = BH // BC
LANES = 16
NQ = D // LANES


def _run(idx_hbm, tok_hbm, pos_hbm, out_hbm,
         idx_v, gidx_v, iflat_v, pos_v, rows_v, out_v, gsem, osem):
    c = lax.axis_index("c")
    s = lax.axis_index("s")
    b0 = pl.multiple_of(c * BH, BH)
    t0 = pl.multiple_of(s * TS, TS)
    tp0 = pl.multiple_of(s * TP, TP)
    pltpu.sync_copy(idx_hbm.at[pl.ds(b0, BH), pl.ds(t0, TS)], idx_v)
    pltpu.sync_copy(pos_hbm.at[pl.ds(tp0, TP), :], pos_v)

    def prep(r, carry):
        for q in range(TS // LANES):
            w = idx_v[r, pl.ds(q * LANES, LANES)]
            gidx_v[r, pl.ds(q * LANES, LANES)] = lax.shift_right_logical(w, 1)
            iflat_v[pl.ds(r * TS + q * LANES, LANES)] = w
        return carry

    lax.fori_loop(0, BH, prep, 0)

    def fire_gathers(ch, slot):
        return [
            pltpu.async_copy(
                tok_hbm.at[gidx_v.at[ch * BC + j]], rows_v.at[slot, j], gsem
            )
            for j in range(BC)
        ]

    ghandles = {0: fire_gathers(0, 0), 1: None}
    ohandles = {0: None, 1: None}

    for ch in range(NCH):
        slot = ch % 2
        if ch + 1 < NCH:
            ghandles[(ch + 1) % 2] = fire_gathers(ch + 1, (ch + 1) % 2)
        for h in ghandles[slot]:
            h.wait()
        if ohandles[slot] is not None:
            ohandles[slot].wait()

        def extract(tl, carry, slot=slot, ch=ch):
            pvs = [pos_v[tl, pl.ds(k * LANES, LANES)] for k in range(2 * NQ)]
            for j in range(BC):
                r = ch * BC + j
                for e in range(2):
                    tt = tl * 2 + e
                    w = iflat_v[pl.ds(r * TS + tt, LANES)][0]
                    base = lax.shift_left(w & 1, 6)
                    for q in range(NQ):
                        val = rows_v[slot, j, tt, pl.ds(base + q * LANES, LANES)]
                        k = e * NQ + q
                        out_v[slot, j, tl, pl.ds(k * LANES, LANES)] = val + pvs[k]
            return carry

        lax.fori_loop(0, TP, extract, 0)
        ohandles[slot] = pltpu.async_copy(
            out_v.at[slot],
            out_hbm.at[pl.ds(b0 + ch * BC, BC), pl.ds(tp0, TP), :],
            osem,
        )
    for slot in range(2):
        if ohandles[slot] is not None:
            ohandles[slot].wait()


def kernel(idx, token_table, position_table):
    mesh = plsc.VectorSubcoreMesh(core_axis_name="c", subcore_axis_name="s")
    run = functools.partial(
        pl.kernel,
        out_type=jax.ShapeDtypeStruct((B, T // 2, 2 * D), jnp.float32),
        mesh=mesh,
        compiler_params=pltpu.CompilerParams(use_tc_tiling_on_sc=True),
        scratch_types=[
            pltpu.VMEM((BH, TS), jnp.int32),
            pltpu.VMEM((BH, TS), jnp.int32),
            pltpu.VMEM((BH * TS + LANES,), jnp.int32),
            pltpu.VMEM((TP, 2 * D), jnp.float32),
            pltpu.VMEM((2, BC, TS, 2 * D), jnp.float32),
            pltpu.VMEM((2, BC, TP, 2 * D), jnp.float32),
            pltpu.SemaphoreType.DMA,
            pltpu.SemaphoreType.DMA,
        ],
    )(_run)
    out_pair = run(
        idx.astype(jnp.int32),
        token_table.reshape(VP, 2 * D),
        position_table.reshape(T // 2, 2 * D),
    )
    return out_pair.reshape(B, T, D)


# speedup vs baseline: 1.1644x; 1.1644x over previous
import functools

import jax
import jax.numpy as jnp
from jax import lax
from jax.experimental import pallas as pl
from jax.experimental.pallas import tpu as pltpu
from jax.experimental.pallas import tpu_sc as plsc

B, T, D = 64, 2048, 64
V = 1000000
NC, NS = 2, 16
TS = 128
TP = TS // 2
BH = B // NC
BC = 2
NCH = BH // BC
LANES = 16
NQ = D // LANES


def _run(idx_hbm, tok_hbm, pos_hbm, out_hbm,
         idx_v, pos_v, rows_v, out_v, gsem, osem):
    c = lax.axis_index("c")
    s = lax.axis_index("s")
    b0 = c * BH
    t0 = s * TS
    tp0 = s * TP
    pltpu.sync_copy(idx_hbm.at[pl.ds(b0, BH), pl.ds(t0, TS)], idx_v)
    pltpu.sync_copy(pos_hbm.at[pl.ds(tp0, TP), :], pos_v)

    def fire_gathers(ch, slot):
        return [
            pltpu.async_copy(
                tok_hbm.at[idx_v.at[ch * BC + j]], rows_v.at[slot, j], gsem
            )
            for j in range(BC)
        ]

    ghandles = {0: fire_gathers(0, 0), 1: None}
    ohandles = {0: None, 1: None}

    for ch in range(NCH):
        slot = ch % 2
        if ch + 1 < NCH:
            ghandles[(ch + 1) % 2] = fire_gathers(ch + 1, (ch + 1) % 2)
        for h in ghandles[slot]:
            h.wait()
        if ohandles[slot] is not None:
            ohandles[slot].wait()

        def extract(tl, carry, slot=slot):
            pvs = [pos_v[tl, pl.ds(k * LANES, LANES)] for k in range(2 * NQ)]
            for j in range(BC):
                for e in range(2):
                    tt = tl * 2 + e
                    for q in range(NQ):
                        val = rows_v[slot, j, tt, pl.ds(q * LANES, LANES)]
                        k = e * NQ + q
                        out_v[slot, j, tl, pl.ds(k * LANES, LANES)] = val + pvs[k]
            return carry

        lax.fori_loop(0, TP, extract, 0)
        ohandles[slot] = pltpu.async_copy(
            out_v.at[slot],
            out_hbm.at[pl.ds(b0 + ch * BC, BC), pl.ds(tp0, TP), :],
            osem,
        )
    for slot in range(2):
        if ohandles[slot] is not None:
            ohandles[slot].wait()


def kernel(idx, token_table, position_table):
    mesh = plsc.VectorSubcoreMesh(core_axis_name="c", subcore_axis_name="s")
    run = functools.partial(
        pl.kernel,
        out_type=jax.ShapeDtypeStruct((B, T // 2, 2 * D), jnp.float32),
        mesh=mesh,
        compiler_params=pltpu.CompilerParams(use_tc_tiling_on_sc=False),
        scratch_types=[
            pltpu.VMEM((BH, TS), jnp.int32),
            pltpu.VMEM((TP, 2 * D), jnp.float32),
            pltpu.VMEM((2, BC, TS, D), jnp.float32),
            pltpu.VMEM((2, BC, TP, 2 * D), jnp.float32),
            pltpu.SemaphoreType.DMA,
            pltpu.SemaphoreType.DMA,
        ],
    )(_run)
    out_pair = run(
        idx.astype(jnp.int32),
        token_table,
        position_table.reshape(T // 2, 2 * D),
    )
    return out_pair.reshape(B, T, D)


# speedup vs baseline: 1.1793x; 1.0128x over previous
import functools

import jax
import jax.numpy as jnp
from jax import lax
from jax.experimental import pallas as pl
from jax.experimental.pallas import tpu as pltpu
from jax.experimental.pallas import tpu_sc as plsc

B, T, D = 64, 2048, 64
V = 1000000
NC, NS = 2, 16
TS = 128
TP = TS // 2
BH = B // NC
BC = 2
NCH = BH // BC
LANES = 16
NQ = D // LANES


def _run(idx_hbm, tok_hbm, pos_hbm, out_hbm,
         idx_v, pos_v, rows_v, out_v, gsem, osem):
    c = lax.axis_index("c")
    s = lax.axis_index("s")
    b0 = c * BH
    t0 = s * TS
    tp0 = s * TP
    pltpu.sync_copy(idx_hbm.at[pl.ds(b0, BH), pl.ds(t0, TS)], idx_v)
    pltpu.sync_copy(pos_hbm.at[pl.ds(tp0, TP), :], pos_v)

    def fire_gathers(ch, slot):
        return [
            pltpu.async_copy(
                tok_hbm.at[idx_v.at[ch * BC + j]], rows_v.at[slot, j], gsem
            )
            for j in range(BC)
        ]

    ghandles = {0: fire_gathers(0, 0), 1: None}
    ohandles = {0: None, 1: None}

    for ch in range(NCH):
        slot = ch % 2
        if ch + 1 < NCH:
            ghandles[(ch + 1) % 2] = fire_gathers(ch + 1, (ch + 1) % 2)
        for h in ghandles[slot]:
            h.wait()
        if ohandles[slot] is not None:
            ohandles[slot].wait()

        def extract(tl, carry, slot=slot):
            pvs = [pos_v[tl, pl.ds(k * LANES, LANES)] for k in range(2 * NQ)]
            for j in range(BC):
                for e in range(2):
                    tt = tl * 2 + e
                    for q in range(NQ):
                        val = rows_v[slot, j, tt, pl.ds(q * LANES, LANES)]
                        k = e * NQ + q
                        out_v[slot, j, tl, pl.ds(k * LANES, LANES)] = val + pvs[k]
            return carry

        lax.fori_loop(0, TP, extract, 0)
        ohandles[slot] = pltpu.async_copy(
            out_v.at[slot],
            out_hbm.at[pl.ds(b0 + ch * BC, BC), pl.ds(tp0, TP), :],
            osem,
        )
    for slot in range(2):
        if ohandles[slot] is not None:
            ohandles[slot].wait()


def kernel(idx, token_table, position_table):
    mesh = plsc.VectorSubcoreMesh(core_axis_name="c", subcore_axis_name="s")
    run = functools.partial(
        pl.kernel,
        out_type=jax.ShapeDtypeStruct((B, T // 2, 2 * D), jnp.float32),
        mesh=mesh,
        compiler_params=pltpu.CompilerParams(use_tc_tiling_on_sc=True),
        scratch_types=[
            pltpu.VMEM((BH, TS), jnp.int32),
            pltpu.VMEM((TP, 2 * D), jnp.float32),
            pltpu.VMEM((2, BC, TS, 2 * D), jnp.float32),
            pltpu.VMEM((2, BC, TP, 2 * D), jnp.float32),
            pltpu.SemaphoreType.DMA,
            pltpu.SemaphoreType.DMA,
        ],
    )(_run)
    out_pair = run(
        idx.astype(jnp.int32),
        jnp.pad(token_table, ((0, 0), (0, D))),
        position_table.reshape(T // 2, 2 * D),
    )
    return out_pair.reshape(B, T, D)
